# baseline (speedup 1.0000x reference)
import functools

import jax
import jax.numpy as jnp
from jax import lax
from jax.experimental import pallas as pl
from jax.experimental.pallas import tpu as pltpu
from jax.experimental.pallas import tpu_sc as plsc


def _build_lookup(BQ, H, V, D, NBUF):
    info = plsc.get_sparse_core_info()
    NC, NS, L = info.num_cores, info.num_subcores, info.num_lanes
    NW = NC * NS
    assert BQ == NW * 128 and D == 32 and L == 16 and H % NBUF == 0
    SUB = D // 8
    mesh = plsc.VectorSubcoreMesh(core_axis_name="c", subcore_axis_name="s")

    @functools.partial(
        pl.kernel,
        mesh=mesh,
        out_type=jax.ShapeDtypeStruct((H, SUB, BQ // 128, 8, 128), jnp.float32),
        scratch_types=(
            [
                pltpu.VMEM((H, 128), jnp.int32),
                pltpu.VMEM((NBUF, 128, D), jnp.float32),
                pltpu.VMEM((NBUF, D, 129), jnp.float32),
            ]
            + [pltpu.SemaphoreType.DMA] * (2 * NBUF)
        ),
        compiler_params=pltpu.CompilerParams(
            use_tc_tiling_on_sc=False, needs_layout_passes=False
        ),
    )
    def lookup_kernel(table_hbm, idxt_hbm, out_hbm, idx_v, rows_v, tr_v, *sems):
        sg = sems[:NBUF]
        sw = sems[NBUF:]
        wid = lax.axis_index("s") * NC + lax.axis_index("c")
        pltpu.sync_copy(idxt_hbm.at[:, pl.ds(wid * 128, 128)], idx_v)

        base_iota = lax.iota(jnp.int32, 16)
        col_lo = base_iota
        col_hi = base_iota + 16

        def start_gather(h, b):
            pltpu.async_copy(
                table_hbm.at[idx_v.at[h]], rows_v.at[b], sg[b]
            )

        for b in range(NBUF):
            start_gather(b, b)

        def outer(t, carry):
            g = t * NBUF
            for b in range(NBUF):
                h = g + b
                pltpu.make_async_copy(
                    table_hbm.at[pl.ds(0, 128)], rows_v.at[b], sg[b]
                ).wait()
                @pl.when(t > 0)
                def _():
                    for a in range(SUB):
                        pltpu.make_async_copy(
                            tr_v.at[b, pl.ds(0, 8), pl.ds(0, 128)],
                            out_hbm.at[0, a, 0],
                            sw[b],
                        ).wait()
                def tr_body(li, carry):
                    for k in range(16):
                        l = li * 16 + k
                        lane = jnp.full((16,), l, jnp.int32)
                        v0 = rows_v[b, l, pl.ds(0, 16)]
                        plsc.store_scatter(tr_v.at[b], [col_lo, lane], v0)
                        v1 = rows_v[b, l, pl.ds(16, 16)]
                        plsc.store_scatter(tr_v.at[b], [col_hi, lane], v1)
                    return carry

                lax.fori_loop(0, 8, tr_body, 0)
                for a in range(SUB):
                    pltpu.async_copy(
                        tr_v.at[b, pl.ds(a * 8, 8), pl.ds(0, 128)],
                        out_hbm.at[h, a, wid],
                        sw[b],
                    )
                nh = h + NBUF

                @pl.when(nh < H)
                def _():
                    start_gather(nh, b)

            return carry

        lax.fori_loop(0, H // NBUF, outer, 0)
        for b in range(NBUF):
            for a in range(SUB):
                pltpu.make_async_copy(
                    tr_v.at[b, pl.ds(0, 8), pl.ds(0, 128)],
                    out_hbm.at[0, a, 0],
                    sw[b],
                ).wait()

    return lookup_kernel


@jax.jit
def kernel(token_idxs, table):
    BQ, H = token_idxs.shape
    V, D = table.shape
    idxt = jnp.swapaxes(token_idxs, 0, 1)
    out5 = _build_lookup(BQ, H, V, D, 4)(table, idxt)
    return out5.transpose((2, 4, 0, 1, 3)).reshape(BQ, H, D)


# speedup vs baseline: 1.0663x; 1.0663x over previous
import functools

import jax
import jax.numpy as jnp
from jax import lax
from jax.experimental import pallas as pl
from jax.experimental.pallas import tpu as pltpu
from jax.experimental.pallas import tpu_sc as plsc


def _build_lookup(BQ, H, V, D, NBUF):
    info = plsc.get_sparse_core_info()
    NC, NS, L = info.num_cores, info.num_subcores, info.num_lanes
    NW = NC * NS
    assert BQ == NW * 128 and D == 32 and L == 16 and H % NBUF == 0
    SUB = D // 8
    mesh = plsc.VectorSubcoreMesh(core_axis_name="c", subcore_axis_name="s")

    @functools.partial(
        pl.kernel,
        mesh=mesh,
        out_type=jax.ShapeDtypeStruct((H, SUB, BQ // 128, 8, 128), jnp.float32),
        scratch_types=(
            [
                pltpu.VMEM((H, 128), jnp.int32),
                pltpu.VMEM((NBUF, 128, D), jnp.float32),
                pltpu.VMEM((NBUF, D, 129), jnp.float32),
            ]
            + [pltpu.SemaphoreType.DMA] * (2 * NBUF)
        ),
        compiler_params=pltpu.CompilerParams(
            use_tc_tiling_on_sc=False, needs_layout_passes=False
        ),
    )
    def lookup_kernel(table_hbm, idxt_hbm, out_hbm, idx_v, rows_v, tr_v, *sems):
        sg = sems[:NBUF]
        sw = sems[NBUF:]
        wid = lax.axis_index("s") * NC + lax.axis_index("c")
        pltpu.sync_copy(idxt_hbm.at[:, pl.ds(wid * 128, 128)], idx_v)

        base_iota = lax.iota(jnp.int32, 16)
        col_lo = base_iota
        col_hi = base_iota + 16

        def start_gather(h, b):
            pltpu.async_copy(
                table_hbm.at[idx_v.at[h]], rows_v.at[b], sg[b]
            )

        for b in range(NBUF):
            start_gather(b, b)

        def outer(t, carry):
            g = t * NBUF
            for b in range(NBUF):
                h = g + b
                pltpu.make_async_copy(
                    table_hbm.at[pl.ds(0, 128)], rows_v.at[b], sg[b]
                ).wait()
                @pl.when(t > 0)
                def _():
                    for a in range(SUB):
                        pltpu.make_async_copy(
                            tr_v.at[b, pl.ds(0, 8), pl.ds(0, 128)],
                            out_hbm.at[0, a, 0],
                            sw[b],
                        ).wait()
                def tr_body(li, carry):
                    for k in range(16):
                        l = li * 16 + k
                        lane = jnp.full((16,), l, jnp.int32)
                        v0 = rows_v[b, l, pl.ds(0, 16)]
                        plsc.store_scatter(tr_v.at[b], [col_lo, lane], v0)
                        v1 = rows_v[b, l, pl.ds(16, 16)]
                        plsc.store_scatter(tr_v.at[b], [col_hi, lane], v1)
                    return carry

                lax.fori_loop(0, 8, tr_body, 0)
                for a in range(SUB):
                    pltpu.async_copy(
                        tr_v.at[b, pl.ds(a * 8, 8), pl.ds(0, 128)],
                        out_hbm.at[h, a, wid],
                        sw[b],
                    )
                nh = h + NBUF

                @pl.when(nh < H)
                def _():
                    start_gather(nh, b)

            return carry

        lax.fori_loop(0, H // NBUF, outer, 0)
        for b in range(NBUF):
            for a in range(SUB):
                pltpu.make_async_copy(
                    tr_v.at[b, pl.ds(0, 8), pl.ds(0, 128)],
                    out_hbm.at[0, a, 0],
                    sw[b],
                ).wait()

    return lookup_kernel


@jax.jit
def kernel(token_idxs, table):
    BQ, H = token_idxs.shape
    V, D = table.shape
    idxt = jnp.swapaxes(token_idxs, 0, 1)
    out5 = _build_lookup(BQ, H, V, D, 2)(table, idxt)
    return out5.transpose((2, 4, 0, 1, 3)).reshape(BQ, H, D)
